# Initial kernel scaffold; baseline (speedup 1.0000x reference)
import jax
import jax.numpy as jnp
from jax.experimental import pallas as pl


def kernel(x, W_rel0, W_root0, b0, W_rel1, W_root1, b1, W_rel2, W_root2, b2, edge_index):
    raise NotImplementedError("write your pallas kernel here")


# speedup vs baseline: 2.8182x; 2.8182x over previous
import functools

import jax
import jax.numpy as jnp
from jax import lax
from jax.experimental import pallas as pl
from jax.experimental.pallas import tpu as pltpu
from jax.experimental.pallas import tpu_sc as plsc

_N = 10000
_E = 320000
_NC, _NS = 2, 16
_NW = _NC * _NS
_EPW = _E // _NW
_C = 128
_NT = 80
_NT_ALLOC = _NT + 2
_NACC = 10240
_RPT = _NACC // _NS
_ZR = 32
_MB = 1000


def _sc_agg(d):
  mesh = plsc.VectorSubcoreMesh(core_axis_name="c", subcore_axis_name="s")

  @functools.partial(
      pl.kernel,
      out_type=jax.ShapeDtypeStruct((_NC, _NACC, d), jnp.float32),
      mesh=mesh,
      scratch_types=[
          pltpu.VMEM((2, _C), jnp.int32),
          pltpu.VMEM((2, _C), jnp.int32),
          pltpu.VMEM((_C, d), jnp.float32),
          pltpu.VMEM((_C, d), jnp.float32),
          pltpu.VMEM((_ZR, d), jnp.float32),
          pltpu.SemaphoreType.DMA,
          pltpu.SemaphoreType.DMA,
          pltpu.SemaphoreType.DMA,
          pltpu.SemaphoreType.DMA,
          pltpu.VMEM_SHARED((_NACC, d), jnp.float32),
      ],
  )
  def agg(y, edges, out, ibuf0, ibuf1, buf0, buf1, zbuf,
          sem_i0, sem_i1, sem_g0, sem_g1, acc):
    c = lax.axis_index("c")
    s = lax.axis_index("s")
    wid = c * _NS + s

    zvec = jnp.zeros((16,), jnp.float32)

    def zrow(i, carry):
      def zcol(j, carry2):
        zbuf[i, pl.ds(j * 16, 16)] = zvec
        return carry2
      return lax.fori_loop(0, d // 16, zcol, carry)

    lax.fori_loop(0, _ZR, zrow, 0)

    def zstripe(k, carry):
      pltpu.sync_copy(zbuf, acc.at[pl.ds(s * _RPT + k * _ZR, _ZR)])
      return carry

    lax.fori_loop(0, _RPT // _ZR, zstripe, 0)
    plsc.subcore_barrier()

    pltpu.async_copy(edges.at[wid, 0], ibuf0, sem_i0)
    pltpu.async_copy(edges.at[wid, 1], ibuf1, sem_i1)
    pltpu.make_async_copy(edges.at[wid, 0], ibuf0, sem_i0).wait()
    pltpu.async_copy(y.at[ibuf0.at[0]], buf0, sem_g0)

    def body(t, carry):
      j = 2 * t
      pltpu.make_async_copy(edges.at[wid, j + 1], ibuf1, sem_i1).wait()
      pltpu.async_copy(y.at[ibuf1.at[0]], buf1, sem_g1)
      pltpu.make_async_copy(y.at[ibuf0.at[0]], buf0, sem_g0).wait()
      pltpu.sync_copy(buf0, acc.at[ibuf0.at[1]], add=True)
      pltpu.async_copy(edges.at[wid, j + 2], ibuf0, sem_i0)
      pltpu.make_async_copy(edges.at[wid, j + 2], ibuf0, sem_i0).wait()
      pltpu.async_copy(y.at[ibuf0.at[0]], buf0, sem_g0)
      pltpu.make_async_copy(y.at[ibuf1.at[0]], buf1, sem_g1).wait()
      pltpu.sync_copy(buf1, acc.at[ibuf1.at[1]], add=True)
      pltpu.async_copy(edges.at[wid, j + 3], ibuf1, sem_i1)
      return carry

    lax.fori_loop(0, _NT // 2, body, 0)
    pltpu.make_async_copy(y.at[ibuf0.at[0]], buf0, sem_g0).wait()
    pltpu.make_async_copy(edges.at[wid, _NT + 1], ibuf1, sem_i1).wait()

    plsc.subcore_barrier()
    pltpu.sync_copy(acc.at[pl.ds(s * _RPT, _RPT)],
                    out.at[c, pl.ds(s * _RPT, _RPT)])

  return agg


_sc_agg128 = _sc_agg(128)


def _linear2(x, wrt, wot, b):
  din, dout = wrt.shape

  def body(x_ref, wr_ref, wo_ref, b_ref, y_ref, r_ref):
    xb = x_ref[...]
    y_ref[...] = jnp.dot(xb, wr_ref[...], preferred_element_type=jnp.float32)
    r_ref[...] = jnp.dot(xb, wo_ref[...],
                         preferred_element_type=jnp.float32) + b_ref[...]

  return pl.pallas_call(
      body,
      grid=(_N // _MB,),
      in_specs=[
          pl.BlockSpec((_MB, din), lambda i: (i, 0)),
          pl.BlockSpec((din, dout), lambda i: (0, 0)),
          pl.BlockSpec((din, dout), lambda i: (0, 0)),
          pl.BlockSpec((1, dout), lambda i: (0, 0)),
      ],
      out_specs=[
          pl.BlockSpec((_MB, dout), lambda i: (i, 0)),
          pl.BlockSpec((_MB, dout), lambda i: (i, 0)),
      ],
      out_shape=[jax.ShapeDtypeStruct((_N, dout), jnp.float32)] * 2,
  )(x, wrt, wot, b)


def _relu_linear2(p, r_prev, wrt, wot, b):
  din, dout = wrt.shape

  def body(p_ref, rp_ref, wr_ref, wo_ref, b_ref, y_ref, r_ref):
    h = jnp.maximum(p_ref[0] + p_ref[1] + rp_ref[...], 0.0)
    y_ref[...] = jnp.dot(h, wr_ref[...], preferred_element_type=jnp.float32)
    r_ref[...] = jnp.dot(h, wo_ref[...],
                         preferred_element_type=jnp.float32) + b_ref[...]

  return pl.pallas_call(
      body,
      grid=(_N // _MB,),
      in_specs=[
          pl.BlockSpec((_NC, _MB, din), lambda i: (0, i, 0)),
          pl.BlockSpec((_MB, din), lambda i: (i, 0)),
          pl.BlockSpec((din, dout), lambda i: (0, 0)),
          pl.BlockSpec((din, dout), lambda i: (0, 0)),
          pl.BlockSpec((1, dout), lambda i: (0, 0)),
      ],
      out_specs=[
          pl.BlockSpec((_MB, dout), lambda i: (i, 0)),
          pl.BlockSpec((_MB, dout), lambda i: (i, 0)),
      ],
      out_shape=[jax.ShapeDtypeStruct((_N, dout), jnp.float32)] * 2,
  )(p, r_prev, wrt, wot, b)


def _tail(p, r_prev):
  dout = r_prev.shape[1]

  def body(p_ref, rp_ref, o_ref):
    o_ref[...] = p_ref[0] + p_ref[1] + rp_ref[...]

  return pl.pallas_call(
      body,
      grid=(_N // _MB,),
      in_specs=[
          pl.BlockSpec((_NC, _MB, dout), lambda i: (0, i, 0)),
          pl.BlockSpec((_MB, dout), lambda i: (i, 0)),
      ],
      out_specs=pl.BlockSpec((_MB, dout), lambda i: (i, 0)),
      out_shape=jax.ShapeDtypeStruct((_N, dout), jnp.float32),
  )(p, r_prev)


def kernel(x, W_rel0, W_root0, b0, W_rel1, W_root1, b1,
           W_rel2, W_root2, b2, edge_index):
  ei = edge_index.astype(jnp.int32)
  pad = _NT_ALLOC * _C - _EPW
  src = jnp.pad(ei[0].reshape(_NW, _EPW), ((0, 0), (0, pad)),
                constant_values=0).reshape(_NW, _NT_ALLOC, _C)
  dst = jnp.pad(ei[1].reshape(_NW, _EPW), ((0, 0), (0, pad)),
                constant_values=_N).reshape(_NW, _NT_ALLOC, _C)
  edges = jnp.stack([src, dst], axis=2)

  b0r = b0.reshape(1, -1)
  b1r = b1.reshape(1, -1)
  wrt2 = jnp.pad(W_rel2.T, ((0, 0), (0, 128 - W_rel2.shape[0])))
  wot2 = jnp.pad(W_root2.T, ((0, 0), (0, 128 - W_root2.shape[0])))
  b2r = jnp.pad(b2, (0, 128 - b2.shape[0])).reshape(1, -1)

  y0, r0 = _linear2(x, W_rel0.T, W_root0.T, b0r)
  p0 = _sc_agg128(y0, edges)
  y1, r1 = _relu_linear2(p0, r0, W_rel1.T, W_root1.T, b1r)
  p1 = _sc_agg128(y1, edges)
  y2, r2 = _relu_linear2(p1, r1, wrt2, wot2, b2r)
  p2 = _sc_agg128(y2, edges)
  out = _tail(p2, r2)
  return out[:, :W_rel2.shape[0]]


# speedup vs baseline: 2.8215x; 1.0012x over previous
import functools

import jax
import jax.numpy as jnp
from jax import lax
from jax.experimental import pallas as pl
from jax.experimental.pallas import tpu as pltpu
from jax.experimental.pallas import tpu_sc as plsc

_N = 10000
_E = 320000
_NC, _NS = 2, 16
_NW = _NC * _NS
_EPW = _E // _NW
_C = 128
_NT = 80
_NT_ALLOC = _NT + 2
_NACC = 10240
_RPT = _NACC // _NS
_ZR = 32
_MB = 1000


def _sc_agg(d):
  mesh = plsc.VectorSubcoreMesh(core_axis_name="c", subcore_axis_name="s")

  @functools.partial(
      pl.kernel,
      out_type=jax.ShapeDtypeStruct((_NC, _NACC, d), jnp.float32),
      mesh=mesh,
      scratch_types=[
          pltpu.VMEM((2, _C), jnp.int32),
          pltpu.VMEM((2, _C), jnp.int32),
          pltpu.VMEM((_C, d), jnp.float32),
          pltpu.VMEM((_C, d), jnp.float32),
          pltpu.VMEM((_ZR, d), jnp.float32),
          pltpu.SemaphoreType.DMA,
          pltpu.SemaphoreType.DMA,
          pltpu.SemaphoreType.DMA,
          pltpu.SemaphoreType.DMA,
          pltpu.SemaphoreType.DMA,
          pltpu.SemaphoreType.DMA,
          pltpu.VMEM_SHARED((_NACC, d), jnp.float32),
      ],
  )
  def agg(y, edges, out, ibuf0, ibuf1, buf0, buf1, zbuf,
          sem_i0, sem_i1, sem_g0, sem_g1, sem_g0b, sem_g1b, acc):

    def fire_gather(ibuf, buf, sa, sb):
      pltpu.async_copy(y.at[ibuf.at[0, pl.ds(0, _C // 2)]],
                       buf.at[pl.ds(0, _C // 2)], sa)
      pltpu.async_copy(y.at[ibuf.at[0, pl.ds(_C // 2, _C // 2)]],
                       buf.at[pl.ds(_C // 2, _C // 2)], sb)

    def wait_gather(ibuf, buf, sa, sb):
      pltpu.make_async_copy(y.at[ibuf.at[0, pl.ds(0, _C // 2)]],
                            buf.at[pl.ds(0, _C // 2)], sa).wait()
      pltpu.make_async_copy(y.at[ibuf.at[0, pl.ds(_C // 2, _C // 2)]],
                            buf.at[pl.ds(_C // 2, _C // 2)], sb).wait()
    c = lax.axis_index("c")
    s = lax.axis_index("s")
    wid = c * _NS + s

    zvec = jnp.zeros((16,), jnp.float32)

    def zrow(i, carry):
      def zcol(j, carry2):
        zbuf[i, pl.ds(j * 16, 16)] = zvec
        return carry2
      return lax.fori_loop(0, d // 16, zcol, carry)

    lax.fori_loop(0, _ZR, zrow, 0)

    def zstripe(k, carry):
      pltpu.sync_copy(zbuf, acc.at[pl.ds(s * _RPT + k * _ZR, _ZR)])
      return carry

    lax.fori_loop(0, _RPT // _ZR, zstripe, 0)
    plsc.subcore_barrier()

    pltpu.async_copy(edges.at[wid, 0], ibuf0, sem_i0)
    pltpu.async_copy(edges.at[wid, 1], ibuf1, sem_i1)
    pltpu.make_async_copy(edges.at[wid, 0], ibuf0, sem_i0).wait()
    fire_gather(ibuf0, buf0, sem_g0, sem_g0b)

    def body(t, carry):
      j = 2 * t
      pltpu.make_async_copy(edges.at[wid, j + 1], ibuf1, sem_i1).wait()
      fire_gather(ibuf1, buf1, sem_g1, sem_g1b)
      wait_gather(ibuf0, buf0, sem_g0, sem_g0b)
      pltpu.sync_copy(buf0, acc.at[ibuf0.at[1]], add=True)
      pltpu.async_copy(edges.at[wid, j + 2], ibuf0, sem_i0)
      pltpu.make_async_copy(edges.at[wid, j + 2], ibuf0, sem_i0).wait()
      fire_gather(ibuf0, buf0, sem_g0, sem_g0b)
      wait_gather(ibuf1, buf1, sem_g1, sem_g1b)
      pltpu.sync_copy(buf1, acc.at[ibuf1.at[1]], add=True)
      pltpu.async_copy(edges.at[wid, j + 3], ibuf1, sem_i1)
      return carry

    lax.fori_loop(0, _NT // 2, body, 0)
    wait_gather(ibuf0, buf0, sem_g0, sem_g0b)
    pltpu.make_async_copy(edges.at[wid, _NT + 1], ibuf1, sem_i1).wait()

    plsc.subcore_barrier()
    pltpu.sync_copy(acc.at[pl.ds(s * _RPT, _RPT)],
                    out.at[c, pl.ds(s * _RPT, _RPT)])

  return agg


_sc_agg128 = _sc_agg(128)


def _linear2(x, wrt, wot, b):
  din, dout = wrt.shape

  def body(x_ref, wr_ref, wo_ref, b_ref, y_ref, r_ref):
    xb = x_ref[...]
    y_ref[...] = jnp.dot(xb, wr_ref[...], preferred_element_type=jnp.float32)
    r_ref[...] = jnp.dot(xb, wo_ref[...],
                         preferred_element_type=jnp.float32) + b_ref[...]

  return pl.pallas_call(
      body,
      grid=(_N // _MB,),
      in_specs=[
          pl.BlockSpec((_MB, din), lambda i: (i, 0)),
          pl.BlockSpec((din, dout), lambda i: (0, 0)),
          pl.BlockSpec((din, dout), lambda i: (0, 0)),
          pl.BlockSpec((1, dout), lambda i: (0, 0)),
      ],
      out_specs=[
          pl.BlockSpec((_MB, dout), lambda i: (i, 0)),
          pl.BlockSpec((_MB, dout), lambda i: (i, 0)),
      ],
      out_shape=[jax.ShapeDtypeStruct((_N, dout), jnp.float32)] * 2,
  )(x, wrt, wot, b)


def _relu_linear2(p, r_prev, wrt, wot, b):
  din, dout = wrt.shape

  def body(p_ref, rp_ref, wr_ref, wo_ref, b_ref, y_ref, r_ref):
    h = jnp.maximum(p_ref[0] + p_ref[1] + rp_ref[...], 0.0)
    y_ref[...] = jnp.dot(h, wr_ref[...], preferred_element_type=jnp.float32)
    r_ref[...] = jnp.dot(h, wo_ref[...],
                         preferred_element_type=jnp.float32) + b_ref[...]

  return pl.pallas_call(
      body,
      grid=(_N // _MB,),
      in_specs=[
          pl.BlockSpec((_NC, _MB, din), lambda i: (0, i, 0)),
          pl.BlockSpec((_MB, din), lambda i: (i, 0)),
          pl.BlockSpec((din, dout), lambda i: (0, 0)),
          pl.BlockSpec((din, dout), lambda i: (0, 0)),
          pl.BlockSpec((1, dout), lambda i: (0, 0)),
      ],
      out_specs=[
          pl.BlockSpec((_MB, dout), lambda i: (i, 0)),
          pl.BlockSpec((_MB, dout), lambda i: (i, 0)),
      ],
      out_shape=[jax.ShapeDtypeStruct((_N, dout), jnp.float32)] * 2,
  )(p, r_prev, wrt, wot, b)


def _tail(p, r_prev):
  dout = r_prev.shape[1]

  def body(p_ref, rp_ref, o_ref):
    o_ref[...] = p_ref[0] + p_ref[1] + rp_ref[...]

  return pl.pallas_call(
      body,
      grid=(_N // _MB,),
      in_specs=[
          pl.BlockSpec((_NC, _MB, dout), lambda i: (0, i, 0)),
          pl.BlockSpec((_MB, dout), lambda i: (i, 0)),
      ],
      out_specs=pl.BlockSpec((_MB, dout), lambda i: (i, 0)),
      out_shape=jax.ShapeDtypeStruct((_N, dout), jnp.float32),
  )(p, r_prev)


def kernel(x, W_rel0, W_root0, b0, W_rel1, W_root1, b1,
           W_rel2, W_root2, b2, edge_index):
  ei = edge_index.astype(jnp.int32)
  pad = _NT_ALLOC * _C - _EPW
  src = jnp.pad(ei[0].reshape(_NW, _EPW), ((0, 0), (0, pad)),
                constant_values=0).reshape(_NW, _NT_ALLOC, _C)
  dst = jnp.pad(ei[1].reshape(_NW, _EPW), ((0, 0), (0, pad)),
                constant_values=_N).reshape(_NW, _NT_ALLOC, _C)
  edges = jnp.stack([src, dst], axis=2)

  b0r = b0.reshape(1, -1)
  b1r = b1.reshape(1, -1)
  wrt2 = jnp.pad(W_rel2.T, ((0, 0), (0, 128 - W_rel2.shape[0])))
  wot2 = jnp.pad(W_root2.T, ((0, 0), (0, 128 - W_root2.shape[0])))
  b2r = jnp.pad(b2, (0, 128 - b2.shape[0])).reshape(1, -1)

  y0, r0 = _linear2(x, W_rel0.T, W_root0.T, b0r)
  p0 = _sc_agg128(y0, edges)
  y1, r1 = _relu_linear2(p0, r0, W_rel1.T, W_root1.T, b1r)
  p1 = _sc_agg128(y1, edges)
  y2, r2 = _relu_linear2(p1, r1, wrt2, wot2, b2r)
  p2 = _sc_agg128(y2, edges)
  out = _tail(p2, r2)
  return out[:, :W_rel2.shape[0]]


# speedup vs baseline: 3.0954x; 1.0970x over previous
import functools

import jax
import jax.numpy as jnp
from jax import lax
from jax.experimental import pallas as pl
from jax.experimental.pallas import tpu as pltpu
from jax.experimental.pallas import tpu_sc as plsc

_N = 10000
_E = 320000
_NC, _NS = 2, 16
_HALF = _N // _NC
_NACC = 5120
_RPT = _NACC // _NS
_YPT = _N // _NS
_EPT = _E // _NS
_C = 32
_NT = 626
_NT_ALLOC = _NT + 2
_MB = 1000


def _sc_agg(d):
  mesh = plsc.VectorSubcoreMesh(core_axis_name="c", subcore_axis_name="s")

  @functools.partial(
      pl.kernel,
      out_type=jax.ShapeDtypeStruct((_NC, _NACC, d), jnp.float32),
      mesh=mesh,
      scratch_types=[
          pltpu.VMEM((2, _C), jnp.int32),
          pltpu.VMEM((2, _C), jnp.int32),
          pltpu.VMEM((_C, d), jnp.float32),
          pltpu.VMEM((_C, d), jnp.float32),
          pltpu.VMEM((8, d), jnp.float32),
          pltpu.SemaphoreType.DMA,
          pltpu.SemaphoreType.DMA,
          pltpu.SemaphoreType.DMA,
          pltpu.SemaphoreType.DMA,
          pltpu.SemaphoreType.DMA,
          pltpu.VMEM_SHARED((_N, d), jnp.float32),
          pltpu.VMEM_SHARED((_NACC, d), jnp.float32),
      ],
  )
  def agg(y, edges, out, ibuf0, ibuf1, buf0, buf1, zbuf,
          sem_i0, sem_i1, sem_g0, sem_g1, sem_s, y_s, acc):
    c = lax.axis_index("c")
    s = lax.axis_index("s")

    @pl.when(s < _N // 1000)
    def _stage():
      sbase = pl.multiple_of(s * 1000, 8)
      pltpu.async_copy(y.at[pl.ds(sbase, 1000)],
                       y_s.at[pl.ds(sbase, 1000)], sem_s)

    zvec = jnp.zeros((16,), jnp.float32)
    for zi in range(8):
      for zj in range(d // 16):
        zbuf[zi, pl.ds(zj * 16, 16)] = zvec

    def zstripe(k, carry):
      pltpu.sync_copy(zbuf, acc.at[pl.ds(s * _RPT + k * 8, 8)])
      return carry

    lax.fori_loop(0, _RPT // 8, zstripe, 0)

    @pl.when(s < _N // 1000)
    def _stage_wait():
      sbase = pl.multiple_of(s * 1000, 8)
      pltpu.make_async_copy(y.at[pl.ds(sbase, 1000)],
                            y_s.at[pl.ds(sbase, 1000)], sem_s).wait()

    plsc.subcore_barrier()

    pltpu.async_copy(edges.at[c, s, 0], ibuf0, sem_i0)
    pltpu.async_copy(edges.at[c, s, 1], ibuf1, sem_i1)
    pltpu.make_async_copy(edges.at[c, s, 0], ibuf0, sem_i0).wait()
    pltpu.async_copy(y_s.at[ibuf0.at[0]], buf0, sem_g0)

    def body(t, carry):
      j = 2 * t
      pltpu.make_async_copy(edges.at[c, s, j + 1], ibuf1, sem_i1).wait()
      pltpu.async_copy(y_s.at[ibuf1.at[0]], buf1, sem_g1)
      pltpu.make_async_copy(y_s.at[ibuf0.at[0]], buf0, sem_g0).wait()
      pltpu.sync_copy(buf0, acc.at[ibuf0.at[1]], add=True)
      pltpu.async_copy(edges.at[c, s, j + 2], ibuf0, sem_i0)
      pltpu.make_async_copy(edges.at[c, s, j + 2], ibuf0, sem_i0).wait()
      pltpu.async_copy(y_s.at[ibuf0.at[0]], buf0, sem_g0)
      pltpu.make_async_copy(y_s.at[ibuf1.at[0]], buf1, sem_g1).wait()
      pltpu.sync_copy(buf1, acc.at[ibuf1.at[1]], add=True)
      pltpu.async_copy(edges.at[c, s, j + 3], ibuf1, sem_i1)
      return carry

    lax.fori_loop(0, _NT // 2, body, 0)
    pltpu.make_async_copy(y_s.at[ibuf0.at[0]], buf0, sem_g0).wait()
    pltpu.make_async_copy(edges.at[c, s, _NT + 1], ibuf1, sem_i1).wait()

    plsc.subcore_barrier()
    pltpu.sync_copy(acc.at[pl.ds(s * _RPT, _RPT)],
                    out.at[c, pl.ds(s * _RPT, _RPT)])

  return agg


_sc_agg128 = _sc_agg(128)


def _linear2(x, wrt, wot, b):
  din, dout = wrt.shape

  def body(x_ref, wr_ref, wo_ref, b_ref, y_ref, r_ref):
    xb = x_ref[...]
    y_ref[...] = jnp.dot(xb, wr_ref[...], preferred_element_type=jnp.float32)
    r_ref[...] = jnp.dot(xb, wo_ref[...],
                         preferred_element_type=jnp.float32) + b_ref[...]

  return pl.pallas_call(
      body,
      grid=(_N // _MB,),
      in_specs=[
          pl.BlockSpec((_MB, din), lambda i: (i, 0)),
          pl.BlockSpec((din, dout), lambda i: (0, 0)),
          pl.BlockSpec((din, dout), lambda i: (0, 0)),
          pl.BlockSpec((1, dout), lambda i: (0, 0)),
      ],
      out_specs=[
          pl.BlockSpec((_MB, dout), lambda i: (i, 0)),
          pl.BlockSpec((_MB, dout), lambda i: (i, 0)),
      ],
      out_shape=[jax.ShapeDtypeStruct((_N, dout), jnp.float32)] * 2,
  )(x, wrt, wot, b)


def _relu_linear2(p, r_prev, wrt, wot, b):
  din, dout = wrt.shape
  hb = _HALF // _MB

  def body(p_ref, rp_ref, wr_ref, wo_ref, b_ref, y_ref, r_ref):
    h = jnp.maximum(p_ref[0] + rp_ref[...], 0.0)
    y_ref[...] = jnp.dot(h, wr_ref[...], preferred_element_type=jnp.float32)
    r_ref[...] = jnp.dot(h, wo_ref[...],
                         preferred_element_type=jnp.float32) + b_ref[...]

  return pl.pallas_call(
      body,
      grid=(_N // _MB,),
      in_specs=[
          pl.BlockSpec((1, _MB, din), lambda i: (i // hb, i % hb, 0)),
          pl.BlockSpec((_MB, din), lambda i: (i, 0)),
          pl.BlockSpec((din, dout), lambda i: (0, 0)),
          pl.BlockSpec((din, dout), lambda i: (0, 0)),
          pl.BlockSpec((1, dout), lambda i: (0, 0)),
      ],
      out_specs=[
          pl.BlockSpec((_MB, dout), lambda i: (i, 0)),
          pl.BlockSpec((_MB, dout), lambda i: (i, 0)),
      ],
      out_shape=[jax.ShapeDtypeStruct((_N, dout), jnp.float32)] * 2,
  )(p, r_prev, wrt, wot, b)


def _tail(p, r_prev):
  dout = r_prev.shape[1]
  hb = _HALF // _MB

  def body(p_ref, rp_ref, o_ref):
    o_ref[...] = p_ref[0] + rp_ref[...]

  return pl.pallas_call(
      body,
      grid=(_N // _MB,),
      in_specs=[
          pl.BlockSpec((1, _MB, dout), lambda i: (i // hb, i % hb, 0)),
          pl.BlockSpec((_MB, dout), lambda i: (i, 0)),
      ],
      out_specs=pl.BlockSpec((_MB, dout), lambda i: (i, 0)),
      out_shape=jax.ShapeDtypeStruct((_N, dout), jnp.float32),
  )(p, r_prev)


def kernel(x, W_rel0, W_root0, b0, W_rel1, W_root1, b1,
           W_rel2, W_root2, b2, edge_index):
  ei = edge_index.astype(jnp.int32)
  pad = _NT_ALLOC * _C - _EPT
  srcp = jnp.pad(ei[0].reshape(_NS, _EPT), ((0, 0), (0, pad)),
                 constant_values=0).reshape(_NS, _NT_ALLOC, _C)
  dstp = jnp.pad(ei[1].reshape(_NS, _EPT), ((0, 0), (0, pad)),
                 constant_values=_N).reshape(_NS, _NT_ALLOC, _C)
  d0 = jnp.where(dstp < _HALF, dstp, _HALF)
  d1 = jnp.where(dstp >= _HALF, dstp - _HALF, _HALF)
  edges = jnp.stack([jnp.stack([srcp, d0], axis=2),
                     jnp.stack([srcp, d1], axis=2)])

  b0r = b0.reshape(1, -1)
  b1r = b1.reshape(1, -1)
  wrt2 = jnp.pad(W_rel2.T, ((0, 0), (0, 128 - W_rel2.shape[0])))
  wot2 = jnp.pad(W_root2.T, ((0, 0), (0, 128 - W_root2.shape[0])))
  b2r = jnp.pad(b2, (0, 128 - b2.shape[0])).reshape(1, -1)

  y0, r0 = _linear2(x, W_rel0.T, W_root0.T, b0r)
  p0 = _sc_agg128(y0, edges)
  y1, r1 = _relu_linear2(p0, r0, W_rel1.T, W_root1.T, b1r)
  p1 = _sc_agg128(y1, edges)
  y2, r2 = _relu_linear2(p1, r1, wrt2, wot2, b2r)
  p2 = _sc_agg128(y2, edges)
  out = _tail(p2, r2)
  return out[:, :W_rel2.shape[0]]


# speedup vs baseline: 4.0672x; 1.3140x over previous
import functools

import jax
import jax.numpy as jnp
from jax import lax
from jax.experimental import pallas as pl
from jax.experimental.pallas import tpu as pltpu
from jax.experimental.pallas import tpu_sc as plsc

_N = 10000
_E = 320000
_NC, _NS = 2, 16
_HALF = _N // _NC
_NACC = 5120
_RPT = _NACC // _NS
_YPT = _N // _NS
_EPT = _E // _NS
_C = 32
_NT = 628
_NT_ALLOC = _NT + 4
_MB = 1000


def _sc_agg(d):
  mesh = plsc.VectorSubcoreMesh(core_axis_name="c", subcore_axis_name="s")

  @functools.partial(
      pl.kernel,
      out_type=jax.ShapeDtypeStruct((_NC, _NACC, d), jnp.float32),
      mesh=mesh,
      scratch_types=[
          pltpu.VMEM((4, 2, _C), jnp.int32),
          pltpu.VMEM((_C, d), jnp.float32),
          pltpu.VMEM((_C, d), jnp.float32),
          pltpu.VMEM((4, d), jnp.float32),
          pltpu.SemaphoreType.DMA,
          pltpu.SemaphoreType.DMA,
          pltpu.SemaphoreType.DMA,
          pltpu.SemaphoreType.DMA,
          pltpu.SemaphoreType.DMA,
          pltpu.SemaphoreType.DMA,
          pltpu.SemaphoreType.DMA,
          pltpu.VMEM_SHARED((_N, d), jnp.float32),
          pltpu.VMEM_SHARED((_NACC, d), jnp.float32),
      ],
  )
  def agg(y, edges, out, ibuf, buf0, buf1, zbuf,
          sem_i0, sem_i1, sem_i2, sem_i3, sem_g0, sem_g1, sem_s, y_s, acc):
    c = lax.axis_index("c")
    s = lax.axis_index("s")

    @pl.when(s < _N // 1000)
    def _stage():
      sbase = pl.multiple_of(s * 1000, 8)
      pltpu.async_copy(y.at[pl.ds(sbase, 1000)],
                       y_s.at[pl.ds(sbase, 1000)], sem_s)

    zvec = jnp.zeros((16,), jnp.float32)
    for zi in range(4):
      for zj in range(d // 16):
        zbuf[zi, pl.ds(zj * 16, 16)] = zvec

    def zstripe(k, carry):
      pltpu.sync_copy(zbuf, acc.at[pl.ds(s * _RPT + k * 4, 4)])
      return carry

    lax.fori_loop(0, _RPT // 4, zstripe, 0)

    @pl.when(s < _N // 1000)
    def _stage_wait():
      sbase = pl.multiple_of(s * 1000, 8)
      pltpu.make_async_copy(y.at[pl.ds(sbase, 1000)],
                            y_s.at[pl.ds(sbase, 1000)], sem_s).wait()

    plsc.subcore_barrier()

    isems = (sem_i0, sem_i1, sem_i2, sem_i3)
    for q in range(4):
      pltpu.async_copy(edges.at[c, s, q], ibuf.at[q], isems[q])

    def body(t, carry):
      j = 4 * t
      bufs = (buf0, buf1)
      gsems = (sem_g0, sem_g1)
      for q in range(4):
        pltpu.make_async_copy(edges.at[c, s, j + q], ibuf.at[q],
                              isems[q]).wait()
        pltpu.async_copy(y_s.at[ibuf.at[q, 0]], bufs[q % 2], gsems[q % 2])
        if q >= 1:
          pq = q - 1
          pltpu.make_async_copy(y_s.at[ibuf.at[pq, 0]], bufs[pq % 2],
                                gsems[pq % 2]).wait()
          pltpu.sync_copy(bufs[pq % 2], acc.at[ibuf.at[pq, 1]], add=True)
          pltpu.async_copy(edges.at[c, s, j + 4 + pq], ibuf.at[pq], isems[pq])
      pltpu.make_async_copy(y_s.at[ibuf.at[3, 0]], buf1, sem_g1).wait()
      pltpu.sync_copy(buf1, acc.at[ibuf.at[3, 1]], add=True)
      pltpu.async_copy(edges.at[c, s, j + 7], ibuf.at[3], isems[3])
      return carry

    lax.fori_loop(0, _NT // 4, body, 0)
    for q in range(4):
      pltpu.make_async_copy(edges.at[c, s, _NT + q], ibuf.at[q],
                            isems[q]).wait()

    plsc.subcore_barrier()
    pltpu.sync_copy(acc.at[pl.ds(s * _RPT, _RPT)],
                    out.at[c, pl.ds(s * _RPT, _RPT)])

  return agg


_sc_agg128 = _sc_agg(128)


def _linear2(x, wrt, wot, b):
  din, dout = wrt.shape

  def body(x_ref, wr_ref, wo_ref, b_ref, y_ref, r_ref):
    xb = x_ref[...]
    y_ref[...] = jnp.dot(xb, wr_ref[...], preferred_element_type=jnp.float32)
    r_ref[...] = jnp.dot(xb, wo_ref[...],
                         preferred_element_type=jnp.float32) + b_ref[...]

  return pl.pallas_call(
      body,
      grid=(_N // _MB,),
      in_specs=[
          pl.BlockSpec((_MB, din), lambda i: (i, 0)),
          pl.BlockSpec((din, dout), lambda i: (0, 0)),
          pl.BlockSpec((din, dout), lambda i: (0, 0)),
          pl.BlockSpec((1, dout), lambda i: (0, 0)),
      ],
      out_specs=[
          pl.BlockSpec((_MB, dout), lambda i: (i, 0)),
          pl.BlockSpec((_MB, dout), lambda i: (i, 0)),
      ],
      out_shape=[jax.ShapeDtypeStruct((_N, dout), jnp.float32)] * 2,
  )(x, wrt, wot, b)


def _relu_linear2(p, r_prev, wrt, wot, b):
  din, dout = wrt.shape
  hb = _HALF // _MB

  def body(p_ref, rp_ref, wr_ref, wo_ref, b_ref, y_ref, r_ref):
    h = jnp.maximum(p_ref[0] + rp_ref[...], 0.0)
    y_ref[...] = jnp.dot(h, wr_ref[...], preferred_element_type=jnp.float32)
    r_ref[...] = jnp.dot(h, wo_ref[...],
                         preferred_element_type=jnp.float32) + b_ref[...]

  return pl.pallas_call(
      body,
      grid=(_N // _MB,),
      in_specs=[
          pl.BlockSpec((1, _MB, din), lambda i: (i // hb, i % hb, 0)),
          pl.BlockSpec((_MB, din), lambda i: (i, 0)),
          pl.BlockSpec((din, dout), lambda i: (0, 0)),
          pl.BlockSpec((din, dout), lambda i: (0, 0)),
          pl.BlockSpec((1, dout), lambda i: (0, 0)),
      ],
      out_specs=[
          pl.BlockSpec((_MB, dout), lambda i: (i, 0)),
          pl.BlockSpec((_MB, dout), lambda i: (i, 0)),
      ],
      out_shape=[jax.ShapeDtypeStruct((_N, dout), jnp.float32)] * 2,
  )(p, r_prev, wrt, wot, b)


def _tail(p, r_prev):
  dout = r_prev.shape[1]
  hb = _HALF // _MB

  def body(p_ref, rp_ref, o_ref):
    o_ref[...] = p_ref[0] + rp_ref[...]

  return pl.pallas_call(
      body,
      grid=(_N // _MB,),
      in_specs=[
          pl.BlockSpec((1, _MB, dout), lambda i: (i // hb, i % hb, 0)),
          pl.BlockSpec((_MB, dout), lambda i: (i, 0)),
      ],
      out_specs=pl.BlockSpec((_MB, dout), lambda i: (i, 0)),
      out_shape=jax.ShapeDtypeStruct((_N, dout), jnp.float32),
  )(p, r_prev)


def kernel(x, W_rel0, W_root0, b0, W_rel1, W_root1, b1,
           W_rel2, W_root2, b2, edge_index):
  ei = edge_index.astype(jnp.int32)
  pad = _NT_ALLOC * _C - _EPT
  srcp = jnp.pad(ei[0].reshape(_NS, _EPT), ((0, 0), (0, pad)),
                 constant_values=0).reshape(_NS, _NT_ALLOC, _C)
  dstp = jnp.pad(ei[1].reshape(_NS, _EPT), ((0, 0), (0, pad)),
                 constant_values=_N).reshape(_NS, _NT_ALLOC, _C)
  d0 = jnp.where(dstp < _HALF, dstp, _HALF)
  d1 = jnp.where(dstp >= _HALF, dstp - _HALF, _HALF)
  edges = jnp.stack([jnp.stack([srcp, d0], axis=2),
                     jnp.stack([srcp, d1], axis=2)])

  b0r = b0.reshape(1, -1)
  b1r = b1.reshape(1, -1)
  wrt2 = jnp.pad(W_rel2.T, ((0, 0), (0, 128 - W_rel2.shape[0])))
  wot2 = jnp.pad(W_root2.T, ((0, 0), (0, 128 - W_root2.shape[0])))
  b2r = jnp.pad(b2, (0, 128 - b2.shape[0])).reshape(1, -1)

  y0, r0 = _linear2(x, W_rel0.T, W_root0.T, b0r)
  p0 = _sc_agg128(y0, edges)
  y1, r1 = _relu_linear2(p0, r0, W_rel1.T, W_root1.T, b1r)
  p1 = _sc_agg128(y1, edges)
  y2, r2 = _relu_linear2(p1, r1, wrt2, wot2, b2r)
  p2 = _sc_agg128(y2, edges)
  out = _tail(p2, r2)
  return out[:, :W_rel2.shape[0]]


# speedup vs baseline: 4.0881x; 1.0052x over previous
import functools

import jax
import jax.numpy as jnp
from jax import lax
from jax.experimental import pallas as pl
from jax.experimental.pallas import tpu as pltpu
from jax.experimental.pallas import tpu_sc as plsc

_N = 10000
_E = 320000
_NC, _NS = 2, 16
_HALF = _N // _NC
_NACC = 5120
_RPT = _NACC // _NS
_YPT = _N // _NS
_EPT = _E // _NS
_C = 32
_NT = 628
_NT_ALLOC = _NT + 4
_MB = 1000


def _sc_agg(d):
  mesh = plsc.VectorSubcoreMesh(core_axis_name="c", subcore_axis_name="s")

  @functools.partial(
      pl.kernel,
      out_type=jax.ShapeDtypeStruct((_NC, _NACC, d), jnp.float32),
      mesh=mesh,
      scratch_types=[
          pltpu.VMEM((4, 2, _C), jnp.int32),
          pltpu.VMEM((_C, d), jnp.float32),
          pltpu.VMEM((_C, d), jnp.float32),
          pltpu.VMEM((4, d), jnp.float32),
          pltpu.SemaphoreType.DMA,
          pltpu.SemaphoreType.DMA,
          pltpu.SemaphoreType.DMA,
          pltpu.SemaphoreType.DMA,
          pltpu.SemaphoreType.DMA,
          pltpu.SemaphoreType.DMA,
          pltpu.SemaphoreType.DMA,
          pltpu.VMEM_SHARED((_N, d), jnp.float32),
          pltpu.VMEM_SHARED((_NACC, d), jnp.float32),
      ],
  )
  def agg(y, edges, out, ibuf, buf0, buf1, zbuf,
          sem_i0, sem_i1, sem_i2, sem_i3, sem_g0, sem_g1, sem_s, y_s, acc):
    c = lax.axis_index("c")
    s = lax.axis_index("s")

    @pl.when(s < _N // 1000)
    def _stage():
      sbase = pl.multiple_of(s * 1000, 8)
      pltpu.async_copy(y.at[pl.ds(sbase, 1000)],
                       y_s.at[pl.ds(sbase, 1000)], sem_s)

    zvec = jnp.zeros((16,), jnp.float32)
    for zi in range(4):
      for zj in range(d // 16):
        zbuf[zi, pl.ds(zj * 16, 16)] = zvec

    def zstripe(k, carry):
      pltpu.sync_copy(zbuf, acc.at[pl.ds(s * _RPT + k * 4, 4)])
      return carry

    lax.fori_loop(0, _RPT // 4, zstripe, 0)

    @pl.when(s < _N // 1000)
    def _stage_wait():
      sbase = pl.multiple_of(s * 1000, 8)
      pltpu.make_async_copy(y.at[pl.ds(sbase, 1000)],
                            y_s.at[pl.ds(sbase, 1000)], sem_s).wait()

    plsc.subcore_barrier()

    isems = (sem_i0, sem_i1, sem_i2, sem_i3)
    for q in range(4):
      pltpu.async_copy(edges.at[c, s, q], ibuf.at[q], isems[q])

    def body(t, carry):
      j = 4 * t
      bufs = (buf0, buf1)
      gsems = (sem_g0, sem_g1)
      for q in range(4):
        pltpu.make_async_copy(edges.at[c, s, j + q], ibuf.at[q],
                              isems[q]).wait()
        pltpu.async_copy(y_s.at[ibuf.at[q, 0]], bufs[q % 2], gsems[q % 2])
        if q >= 1:
          pq = q - 1
          pltpu.make_async_copy(y_s.at[ibuf.at[pq, 0]], bufs[pq % 2],
                                gsems[pq % 2]).wait()
          pltpu.sync_copy(bufs[pq % 2], acc.at[ibuf.at[pq, 1]], add=True)
          pltpu.async_copy(edges.at[c, s, j + 4 + pq], ibuf.at[pq], isems[pq])
      pltpu.make_async_copy(y_s.at[ibuf.at[3, 0]], buf1, sem_g1).wait()
      pltpu.sync_copy(buf1, acc.at[ibuf.at[3, 1]], add=True)
      pltpu.async_copy(edges.at[c, s, j + 7], ibuf.at[3], isems[3])
      return carry

    lax.fori_loop(0, _NT // 4, body, 0)
    for q in range(4):
      pltpu.make_async_copy(edges.at[c, s, _NT + q], ibuf.at[q],
                            isems[q]).wait()

    plsc.subcore_barrier()
    pltpu.sync_copy(acc.at[pl.ds(s * _RPT, _RPT)],
                    out.at[c, pl.ds(s * _RPT, _RPT)])

  return agg


_sc_agg128 = _sc_agg(128)


def _linear1(x, w, b=None):
  din, dout = w.shape

  if b is None:
    def body(x_ref, w_ref, o_ref):
      o_ref[...] = jnp.dot(x_ref[...], w_ref[...],
                           preferred_element_type=jnp.float32)
    args, specs = (x, w), []
  else:
    def body(x_ref, w_ref, b_ref, o_ref):
      o_ref[...] = jnp.dot(x_ref[...], w_ref[...],
                           preferred_element_type=jnp.float32) + b_ref[...]
    args, specs = (x, w, b), [pl.BlockSpec((1, dout), lambda i: (0, 0))]

  return pl.pallas_call(
      body,
      grid=(_N // _MB,),
      in_specs=[
          pl.BlockSpec((_MB, din), lambda i: (i, 0)),
          pl.BlockSpec((din, dout), lambda i: (0, 0)),
      ] + specs,
      out_specs=pl.BlockSpec((_MB, dout), lambda i: (i, 0)),
      out_shape=jax.ShapeDtypeStruct((_N, dout), jnp.float32),
  )(*args)


def _relu_linear1(p, r_prev, w, b=None):
  din, dout = w.shape
  hb = _HALF // _MB

  if b is None:
    def body(p_ref, rp_ref, w_ref, o_ref):
      h = jnp.maximum(p_ref[0] + rp_ref[...], 0.0)
      o_ref[...] = jnp.dot(h, w_ref[...], preferred_element_type=jnp.float32)
    args, specs = (p, r_prev, w), []
  else:
    def body(p_ref, rp_ref, w_ref, b_ref, o_ref):
      h = jnp.maximum(p_ref[0] + rp_ref[...], 0.0)
      o_ref[...] = jnp.dot(h, w_ref[...],
                           preferred_element_type=jnp.float32) + b_ref[...]
    args, specs = (p, r_prev, w, b), [pl.BlockSpec((1, dout), lambda i: (0, 0))]

  return pl.pallas_call(
      body,
      grid=(_N // _MB,),
      in_specs=[
          pl.BlockSpec((1, _MB, din), lambda i: (i // hb, i % hb, 0)),
          pl.BlockSpec((_MB, din), lambda i: (i, 0)),
          pl.BlockSpec((din, dout), lambda i: (0, 0)),
      ] + specs,
      out_specs=pl.BlockSpec((_MB, dout), lambda i: (i, 0)),
      out_shape=jax.ShapeDtypeStruct((_N, dout), jnp.float32),
  )(*args)


def _tail(p, r_prev):
  dout = r_prev.shape[1]
  hb = _HALF // _MB

  def body(p_ref, rp_ref, o_ref):
    o_ref[...] = p_ref[0] + rp_ref[...]

  return pl.pallas_call(
      body,
      grid=(_N // _MB,),
      in_specs=[
          pl.BlockSpec((1, _MB, dout), lambda i: (i // hb, i % hb, 0)),
          pl.BlockSpec((_MB, dout), lambda i: (i, 0)),
      ],
      out_specs=pl.BlockSpec((_MB, dout), lambda i: (i, 0)),
      out_shape=jax.ShapeDtypeStruct((_N, dout), jnp.float32),
  )(p, r_prev)


def kernel(x, W_rel0, W_root0, b0, W_rel1, W_root1, b1,
           W_rel2, W_root2, b2, edge_index):
  ei = edge_index.astype(jnp.int32)
  pad = _NT_ALLOC * _C - _EPT
  srcp = jnp.pad(ei[0].reshape(_NS, _EPT), ((0, 0), (0, pad)),
                 constant_values=0).reshape(_NS, _NT_ALLOC, _C)
  dstp = jnp.pad(ei[1].reshape(_NS, _EPT), ((0, 0), (0, pad)),
                 constant_values=_N).reshape(_NS, _NT_ALLOC, _C)
  d0 = jnp.where(dstp < _HALF, dstp, _HALF)
  d1 = jnp.where(dstp >= _HALF, dstp - _HALF, _HALF)
  edges = jnp.stack([jnp.stack([srcp, d0], axis=2),
                     jnp.stack([srcp, d1], axis=2)])

  b0r = b0.reshape(1, -1)
  b1r = b1.reshape(1, -1)
  wrt2 = jnp.pad(W_rel2.T, ((0, 0), (0, 128 - W_rel2.shape[0])))
  wot2 = jnp.pad(W_root2.T, ((0, 0), (0, 128 - W_root2.shape[0])))
  b2r = jnp.pad(b2, (0, 128 - b2.shape[0])).reshape(1, -1)

  y0 = _linear1(x, W_rel0.T)
  p0 = _sc_agg128(y0, edges)
  r0 = _linear1(x, W_root0.T, b0r)
  y1 = _relu_linear1(p0, r0, W_rel1.T)
  p1 = _sc_agg128(y1, edges)
  r1 = _relu_linear1(p0, r0, W_root1.T, b1r)
  y2 = _relu_linear1(p1, r1, wrt2)
  p2 = _sc_agg128(y2, edges)
  r2 = _relu_linear1(p1, r1, wot2, b2r)
  out = _tail(p2, r2)
  return out[:, :W_rel2.shape[0]]
